# baseline (device time: 31598 ns/iter reference)
import jax
import jax.numpy as jnp
from jax import lax
from jax.experimental import pallas as pl
from jax.experimental.pallas import tpu as pltpu

N_DEV = 4
B = 2
S_SH = 256
HQ = 4
DH = 64
BH = B * HQ
DM = 512
HD = HQ * DH

G = 32
HALO = 128
W = G + HALO + S_SH + HALO
SENT = 100000

C_G, C_L, C_O, C_R = 0, G, G + HALO, G + HALO + S_SH


def kernel(x, Wq, K_ext, V_ext, Wo):
    Kt = jnp.transpose(K_ext, (0, 2, 1, 3)).reshape(BH, S_SH, DH)
    Vt = jnp.transpose(V_ext, (0, 2, 1, 3)).reshape(BH, S_SH, DH)

    def body(x_ref, wq_ref, k_ref, v_ref, wo_ref, out_ref,
             kpack, vpack, qscr, q0buf, dbuf, drecv, locstash, ctxscr,
             lsem, bsend, brecv, gsend, grecv, qsend, qrecv, dsend, dsem):
        my = lax.axis_index("i")
        right = lax.rem(my + 1, N_DEV)
        left = lax.rem(my + 3, N_DEV)

        vpack[:, C_G:C_L, :] = jnp.zeros((BH, G, DH), jnp.float32)
        vpack[:, C_L:C_O, :] = jnp.zeros((BH, HALO, DH), jnp.float32)
        vpack[:, C_R:W, :] = jnp.zeros((BH, HALO, DH), jnp.float32)

        barrier_sem = pltpu.get_barrier_semaphore()
        for ofs in (1, 2, 3):
            pl.semaphore_signal(
                barrier_sem, inc=1,
                device_id=(lax.rem(my + ofs, N_DEV),),
                device_id_type=pl.DeviceIdType.MESH,
            )
        pl.semaphore_wait(barrier_sem, 3)

        cpk = pltpu.make_async_copy(k_ref, kpack.at[:, C_O:C_R, :], lsem.at[0])
        cpv = pltpu.make_async_copy(v_ref, vpack.at[:, C_O:C_R, :], lsem.at[1])
        cpk.start()
        cpv.start()

        @pl.when(my != N_DEV - 1)
        def _():
            for sem_i, (src, dstbuf) in enumerate(
                    [(k_ref, kpack), (v_ref, vpack)]):
                pltpu.make_async_remote_copy(
                    src_ref=src.at[:, S_SH - HALO:S_SH, :],
                    dst_ref=dstbuf.at[:, C_L:C_O, :],
                    send_sem=bsend.at[sem_i],
                    recv_sem=brecv.at[sem_i],
                    device_id=(right,),
                    device_id_type=pl.DeviceIdType.MESH,
                ).start()

        @pl.when(my != 0)
        def _():
            for sem_i, (src, dstbuf) in enumerate(
                    [(k_ref, kpack), (v_ref, vpack)]):
                pltpu.make_async_remote_copy(
                    src_ref=src.at[:, 0:HALO, :],
                    dst_ref=dstbuf.at[:, C_R:W, :],
                    send_sem=bsend.at[2 + sem_i],
                    recv_sem=brecv.at[2 + sem_i],
                    device_id=(left,),
                    device_id_type=pl.DeviceIdType.MESH,
                ).start()

        @pl.when(my == 0)
        def _():
            for d in (1, 2, 3):
                for sem_i, (src, dstbuf) in enumerate(
                        [(k_ref, kpack), (v_ref, vpack)]):
                    pltpu.make_async_remote_copy(
                        src_ref=src.at[:, 0:G, :],
                        dst_ref=dstbuf.at[:, C_G:C_L, :],
                        send_sem=gsend.at[2 * (d - 1) + sem_i],
                        recv_sem=grecv.at[sem_i],
                        device_id=(d,),
                        device_id_type=pl.DeviceIdType.MESH,
                    ).start()

        for b in range(B):
            qscr[b] = jnp.dot(x_ref[b], wq_ref[...],
                              preferred_element_type=jnp.float32)

        @pl.when(my == 0)
        def _():
            for d in (1, 2, 3):
                pltpu.make_async_remote_copy(
                    src_ref=qscr.at[:, 0:G, :],
                    dst_ref=q0buf,
                    send_sem=qsend.at[d - 1],
                    recv_sem=qrecv.at[0],
                    device_id=(d,),
                    device_id_type=pl.DeviceIdType.MESH,
                ).start()

        @pl.when(my != 0)
        def _():
            pltpu.make_async_remote_copy(
                src_ref=qscr.at[:, 0:G, :], dst_ref=q0buf,
                send_sem=qsend.at[0], recv_sem=qrecv.at[0],
                device_id=(0,), device_id_type=pl.DeviceIdType.MESH,
            ).wait_recv()
            for b in range(B):
                for h in range(HQ):
                    bh = b * HQ + h
                    qd = q0buf[b, :, h * DH:(h + 1) * DH]
                    sc = lax.dot_general(
                        qd, k_ref[bh], (((1,), (1,)), ((), ())),
                        preferred_element_type=jnp.float32) * 0.125
                    md = jnp.max(sc, axis=1, keepdims=True)
                    e = jnp.exp(sc - md)
                    sd = jnp.sum(e, axis=1, keepdims=True)
                    cd = jnp.dot(e, v_ref[bh],
                                 preferred_element_type=jnp.float32)
                    dbuf[bh] = jnp.concatenate(
                        [cd, md, sd, jnp.zeros((G, 62), jnp.float32)], axis=1)
            pltpu.make_async_remote_copy(
                src_ref=dbuf, dst_ref=drecv.at[my - 1],
                send_sem=dsend.at[0], recv_sem=dsem.at[my - 1],
                device_id=(0,), device_id_type=pl.DeviceIdType.MESH,
            ).start()

        cpk.wait()
        cpv.wait()

        @pl.when(my != 0)
        def _():
            for sem_i, (src, dstbuf) in enumerate(
                    [(k_ref, kpack), (v_ref, vpack)]):
                pltpu.make_async_remote_copy(
                    src_ref=src.at[:, S_SH - HALO:S_SH, :],
                    dst_ref=dstbuf.at[:, C_L:C_O, :],
                    send_sem=bsend.at[sem_i], recv_sem=brecv.at[sem_i],
                    device_id=(left,), device_id_type=pl.DeviceIdType.MESH,
                ).wait_recv()
                pltpu.make_async_remote_copy(
                    src_ref=src.at[:, 0:G, :],
                    dst_ref=dstbuf.at[:, C_G:C_L, :],
                    send_sem=gsend.at[sem_i], recv_sem=grecv.at[sem_i],
                    device_id=(0,), device_id_type=pl.DeviceIdType.MESH,
                ).wait_recv()

        @pl.when(my != N_DEV - 1)
        def _():
            for sem_i, (src, dstbuf) in enumerate(
                    [(k_ref, kpack), (v_ref, vpack)]):
                pltpu.make_async_remote_copy(
                    src_ref=src.at[:, 0:HALO, :],
                    dst_ref=dstbuf.at[:, C_R:W, :],
                    send_sem=bsend.at[2 + sem_i], recv_sem=brecv.at[2 + sem_i],
                    device_id=(right,), device_id_type=pl.DeviceIdType.MESH,
                ).wait_recv()

        qg = lax.broadcasted_iota(jnp.int32, (S_SH, W), 0) + my * S_SH
        ci = lax.broadcasted_iota(jnp.int32, (S_SH, W), 1)
        gk = jnp.where(
            ci < C_L, ci + jnp.where(my == 0, SENT, 0),
            jnp.where(
                ci < C_O, ci - C_L + jnp.where(my == 0, SENT,
                                               my * S_SH - HALO),
                jnp.where(
                    ci < C_R, ci - C_O + my * S_SH,
                    ci - C_R + jnp.where(my == N_DEV - 1, SENT,
                                         my * S_SH + S_SH))))
        mask = ((jnp.abs(qg - gk) <= HALO) | (gk < G)
                | ((qg < G) & (gk < S_SH)))
        neg = jnp.float32(-1e9)

        for b in range(B):
            for h in range(HQ):
                bh = b * HQ + h
                qh = qscr[b, :, h * DH:(h + 1) * DH]
                scores = lax.dot_general(
                    qh, kpack[bh], (((1,), (1,)), ((), ())),
                    preferred_element_type=jnp.float32) * 0.125
                scores = jnp.where(mask, scores, neg)
                m = jnp.max(scores, axis=1, keepdims=True)
                e = jnp.exp(scores - m)
                s = jnp.sum(e, axis=1, keepdims=True)
                cu = jnp.dot(e, vpack[bh],
                             preferred_element_type=jnp.float32)
                ctxscr[b, :, h * DH:(h + 1) * DH] = cu / s
                locstash[bh] = jnp.concatenate(
                    [cu[0:G], m[0:G], s[0:G],
                     jnp.zeros((G, 62), jnp.float32)], axis=1)

        @pl.when(my == 0)
        def _():
            for d in range(3):
                pltpu.make_async_remote_copy(
                    src_ref=dbuf, dst_ref=drecv.at[d],
                    send_sem=dsend.at[0], recv_sem=dsem.at[d],
                    device_id=(0,), device_id_type=pl.DeviceIdType.MESH,
                ).wait_recv()
            for b in range(B):
                for h in range(HQ):
                    bh = b * HQ + h
                    parts = [locstash[bh]] + [drecv[d, bh] for d in range(3)]
                    ms = [p[:, DH:DH + 1] for p in parts]
                    M = jnp.maximum(jnp.maximum(ms[0], ms[1]),
                                    jnp.maximum(ms[2], ms[3]))
                    S = jnp.float32(0)
                    C = jnp.float32(0)
                    for p, mi in zip(parts, ms):
                        w = jnp.exp(mi - M)
                        S = S + p[:, DH + 1:DH + 2] * w
                        C = C + p[:, 0:DH] * w
                    ctxscr[b, 0:G, h * DH:(h + 1) * DH] = C / S

        for b in range(B):
            out_ref[b] = jnp.dot(ctxscr[b], wo_ref[...],
                                 preferred_element_type=jnp.float32)

        @pl.when(my != N_DEV - 1)
        def _():
            for sem_i, src in enumerate([k_ref, v_ref]):
                pltpu.make_async_remote_copy(
                    src_ref=src.at[:, S_SH - HALO:S_SH, :],
                    dst_ref=kpack.at[:, C_L:C_O, :],
                    send_sem=bsend.at[sem_i], recv_sem=brecv.at[sem_i],
                    device_id=(right,), device_id_type=pl.DeviceIdType.MESH,
                ).wait_send()

        @pl.when(my != 0)
        def _():
            for sem_i, src in enumerate([k_ref, v_ref]):
                pltpu.make_async_remote_copy(
                    src_ref=src.at[:, 0:HALO, :],
                    dst_ref=kpack.at[:, C_R:W, :],
                    send_sem=bsend.at[2 + sem_i], recv_sem=brecv.at[2 + sem_i],
                    device_id=(left,), device_id_type=pl.DeviceIdType.MESH,
                ).wait_send()
            pltpu.make_async_remote_copy(
                src_ref=dbuf, dst_ref=drecv.at[0],
                send_sem=dsend.at[0], recv_sem=dsem.at[0],
                device_id=(0,), device_id_type=pl.DeviceIdType.MESH,
            ).wait_send()

        @pl.when(my == 0)
        def _():
            for d in (1, 2, 3):
                for sem_i, src in enumerate([k_ref, v_ref]):
                    pltpu.make_async_remote_copy(
                        src_ref=src.at[:, 0:G, :],
                        dst_ref=kpack.at[:, C_G:C_L, :],
                        send_sem=gsend.at[2 * (d - 1) + sem_i],
                        recv_sem=grecv.at[sem_i],
                        device_id=(d,), device_id_type=pl.DeviceIdType.MESH,
                    ).wait_send()
                pltpu.make_async_remote_copy(
                    src_ref=qscr.at[:, 0:G, :], dst_ref=q0buf,
                    send_sem=qsend.at[d - 1], recv_sem=qrecv.at[0],
                    device_id=(d,), device_id_type=pl.DeviceIdType.MESH,
                ).wait_send()

        import functools

        @functools.partial(pl.run_scoped,
                           second_barrier=pltpu.SemaphoreType.REGULAR)
        def _(second_barrier):
            for ofs in (1, 2, 3):
                pl.semaphore_signal(
                    second_barrier, inc=1,
                    device_id=(lax.rem(my + ofs, N_DEV),),
                    device_id_type=pl.DeviceIdType.MESH,
                )
            pl.semaphore_wait(second_barrier, 3)

    return pl.pallas_call(
        body,
        out_shape=jax.ShapeDtypeStruct((B, S_SH, DM), jnp.float32),
        in_specs=[pl.BlockSpec(memory_space=pltpu.VMEM)] * 5,
        out_specs=pl.BlockSpec(memory_space=pltpu.VMEM),
        scratch_shapes=[
            pltpu.VMEM((BH, W, DH), jnp.float32),
            pltpu.VMEM((BH, W, DH), jnp.float32),
            pltpu.VMEM((B, S_SH, HD), jnp.float32),
            pltpu.VMEM((B, G, HD), jnp.float32),
            pltpu.VMEM((BH, G, 128), jnp.float32),
            pltpu.VMEM((3, BH, G, 128), jnp.float32),
            pltpu.VMEM((BH, G, 128), jnp.float32),
            pltpu.VMEM((B, S_SH, HD), jnp.float32),
            pltpu.SemaphoreType.DMA((2,)),
            pltpu.SemaphoreType.DMA((4,)),
            pltpu.SemaphoreType.DMA((4,)),
            pltpu.SemaphoreType.DMA((6,)),
            pltpu.SemaphoreType.DMA((2,)),
            pltpu.SemaphoreType.DMA((3,)),
            pltpu.SemaphoreType.DMA((1,)),
            pltpu.SemaphoreType.DMA((1,)),
            pltpu.SemaphoreType.DMA((3,)),
        ],
        compiler_params=pltpu.CompilerParams(collective_id=0),
    )(x, Wq, Kt, Vt, Wo)


# device time: 24298 ns/iter; 1.3004x vs baseline; 1.3004x over previous
import jax
import jax.numpy as jnp
from jax import lax
from jax.experimental import pallas as pl
from jax.experimental.pallas import tpu as pltpu

N_DEV = 4
B = 2
S_SH = 256
HQ = 4
DH = 64
BH = B * HQ
DM = 512
HD = HQ * DH

G = 32
HALO = 128
W = G + HALO + S_SH + HALO
SENT = 100000

C_G, C_L, C_O, C_R = 0, G, G + HALO, G + HALO + S_SH


def kernel(x, Wq, K_ext, V_ext, Wo):
    Kt = jnp.transpose(K_ext, (0, 2, 1, 3)).reshape(BH, S_SH, DH)
    Kt = Kt.astype(jnp.bfloat16)
    Vt = jnp.transpose(V_ext, (0, 2, 1, 3)).reshape(BH, S_SH, DH)
    Vt = Vt.astype(jnp.bfloat16)

    def body(x_ref, wq_ref, k_ref, v_ref, wo_ref, out_ref,
             kpack, vpack, qscr, q0buf, dbuf, drecv, locstash, ctxscr,
             lsem, bsend, brecv, gsend, grecv, qsend, qrecv, dsend, dsem):
        my = lax.axis_index("i")
        right = lax.rem(my + 1, N_DEV)
        left = lax.rem(my + 3, N_DEV)

        vpack[:, C_G:C_L, :] = jnp.zeros((BH, G, DH), jnp.bfloat16)
        vpack[:, C_L:C_O, :] = jnp.zeros((BH, HALO, DH), jnp.bfloat16)
        vpack[:, C_R:W, :] = jnp.zeros((BH, HALO, DH), jnp.bfloat16)

        barrier_sem = pltpu.get_barrier_semaphore()
        for ofs in (1, 2, 3):
            pl.semaphore_signal(
                barrier_sem, inc=1,
                device_id=(lax.rem(my + ofs, N_DEV),),
                device_id_type=pl.DeviceIdType.MESH,
            )
        pl.semaphore_wait(barrier_sem, 3)

        cpk = pltpu.make_async_copy(k_ref, kpack.at[:, C_O:C_R, :], lsem.at[0])
        cpv = pltpu.make_async_copy(v_ref, vpack.at[:, C_O:C_R, :], lsem.at[1])
        cpk.start()
        cpv.start()

        @pl.when(my != N_DEV - 1)
        def _():
            for sem_i, (src, dstbuf) in enumerate(
                    [(k_ref, kpack), (v_ref, vpack)]):
                pltpu.make_async_remote_copy(
                    src_ref=src.at[:, S_SH - HALO:S_SH, :],
                    dst_ref=dstbuf.at[:, C_L:C_O, :],
                    send_sem=bsend.at[sem_i],
                    recv_sem=brecv.at[sem_i],
                    device_id=(right,),
                    device_id_type=pl.DeviceIdType.MESH,
                ).start()

        @pl.when(my != 0)
        def _():
            for sem_i, (src, dstbuf) in enumerate(
                    [(k_ref, kpack), (v_ref, vpack)]):
                pltpu.make_async_remote_copy(
                    src_ref=src.at[:, 0:HALO, :],
                    dst_ref=dstbuf.at[:, C_R:W, :],
                    send_sem=bsend.at[2 + sem_i],
                    recv_sem=brecv.at[2 + sem_i],
                    device_id=(left,),
                    device_id_type=pl.DeviceIdType.MESH,
                ).start()

        @pl.when(my == 0)
        def _():
            for d in (1, 2, 3):
                for sem_i, (src, dstbuf) in enumerate(
                        [(k_ref, kpack), (v_ref, vpack)]):
                    pltpu.make_async_remote_copy(
                        src_ref=src.at[:, 0:G, :],
                        dst_ref=dstbuf.at[:, C_G:C_L, :],
                        send_sem=gsend.at[2 * (d - 1) + sem_i],
                        recv_sem=grecv.at[sem_i],
                        device_id=(d,),
                        device_id_type=pl.DeviceIdType.MESH,
                    ).start()

        for b in range(B):
            qscr[b] = jnp.dot(x_ref[b], wq_ref[...],
                              preferred_element_type=jnp.float32)

        @pl.when(my == 0)
        def _():
            for d in (1, 2, 3):
                pltpu.make_async_remote_copy(
                    src_ref=qscr.at[:, 0:G, :],
                    dst_ref=q0buf,
                    send_sem=qsend.at[d - 1],
                    recv_sem=qrecv.at[0],
                    device_id=(d,),
                    device_id_type=pl.DeviceIdType.MESH,
                ).start()

        @pl.when(my != 0)
        def _():
            pltpu.make_async_remote_copy(
                src_ref=qscr.at[:, 0:G, :], dst_ref=q0buf,
                send_sem=qsend.at[0], recv_sem=qrecv.at[0],
                device_id=(0,), device_id_type=pl.DeviceIdType.MESH,
            ).wait_recv()
            for b in range(B):
                for h in range(HQ):
                    bh = b * HQ + h
                    qd = q0buf[b, :, h * DH:(h + 1) * DH]
                    sc = lax.dot_general(
                        qd.astype(jnp.bfloat16), k_ref[bh],
                        (((1,), (1,)), ((), ())),
                        preferred_element_type=jnp.float32) * 0.125
                    md = jnp.max(sc, axis=1, keepdims=True)
                    e = jnp.exp(sc - md)
                    sd = jnp.sum(e, axis=1, keepdims=True)
                    cd = jnp.dot(e.astype(jnp.bfloat16), v_ref[bh],
                                 preferred_element_type=jnp.float32)
                    dbuf[bh] = jnp.concatenate(
                        [cd, md, sd, jnp.zeros((G, 62), jnp.float32)], axis=1)
            pltpu.make_async_remote_copy(
                src_ref=dbuf, dst_ref=drecv.at[my - 1],
                send_sem=dsend.at[0], recv_sem=dsem.at[my - 1],
                device_id=(0,), device_id_type=pl.DeviceIdType.MESH,
            ).start()

        cpk.wait()
        cpv.wait()

        @pl.when(my != 0)
        def _():
            for sem_i, (src, dstbuf) in enumerate(
                    [(k_ref, kpack), (v_ref, vpack)]):
                pltpu.make_async_remote_copy(
                    src_ref=src.at[:, S_SH - HALO:S_SH, :],
                    dst_ref=dstbuf.at[:, C_L:C_O, :],
                    send_sem=bsend.at[sem_i], recv_sem=brecv.at[sem_i],
                    device_id=(left,), device_id_type=pl.DeviceIdType.MESH,
                ).wait_recv()
                pltpu.make_async_remote_copy(
                    src_ref=src.at[:, 0:G, :],
                    dst_ref=dstbuf.at[:, C_G:C_L, :],
                    send_sem=gsend.at[sem_i], recv_sem=grecv.at[sem_i],
                    device_id=(0,), device_id_type=pl.DeviceIdType.MESH,
                ).wait_recv()

        @pl.when(my != N_DEV - 1)
        def _():
            for sem_i, (src, dstbuf) in enumerate(
                    [(k_ref, kpack), (v_ref, vpack)]):
                pltpu.make_async_remote_copy(
                    src_ref=src.at[:, 0:HALO, :],
                    dst_ref=dstbuf.at[:, C_R:W, :],
                    send_sem=bsend.at[2 + sem_i], recv_sem=brecv.at[2 + sem_i],
                    device_id=(right,), device_id_type=pl.DeviceIdType.MESH,
                ).wait_recv()

        qg = lax.broadcasted_iota(jnp.int32, (S_SH, W), 0) + my * S_SH
        ci = lax.broadcasted_iota(jnp.int32, (S_SH, W), 1)
        gk = jnp.where(
            ci < C_L, ci + jnp.where(my == 0, SENT, 0),
            jnp.where(
                ci < C_O, ci - C_L + jnp.where(my == 0, SENT,
                                               my * S_SH - HALO),
                jnp.where(
                    ci < C_R, ci - C_O + my * S_SH,
                    ci - C_R + jnp.where(my == N_DEV - 1, SENT,
                                         my * S_SH + S_SH))))
        mask = ((jnp.abs(qg - gk) <= HALO) | (gk < G)
                | ((qg < G) & (gk < S_SH)))
        neg = jnp.float32(-1e9)

        for b in range(B):
            for h in range(HQ):
                bh = b * HQ + h
                qh = qscr[b, :, h * DH:(h + 1) * DH]
                scores = lax.dot_general(
                    qh.astype(jnp.bfloat16), kpack[bh],
                    (((1,), (1,)), ((), ())),
                    preferred_element_type=jnp.float32) * 0.125
                scores = jnp.where(mask, scores, neg)
                m = jnp.max(scores, axis=1, keepdims=True)
                e = jnp.exp(scores - m)
                s = jnp.sum(e, axis=1, keepdims=True)
                cu = jnp.dot(e.astype(jnp.bfloat16), vpack[bh],
                             preferred_element_type=jnp.float32)
                ctxscr[b, :, h * DH:(h + 1) * DH] = cu / s
                locstash[bh] = jnp.concatenate(
                    [cu[0:G], m[0:G], s[0:G],
                     jnp.zeros((G, 62), jnp.float32)], axis=1)

        @pl.when(my == 0)
        def _():
            for d in range(3):
                pltpu.make_async_remote_copy(
                    src_ref=dbuf, dst_ref=drecv.at[d],
                    send_sem=dsend.at[0], recv_sem=dsem.at[d],
                    device_id=(0,), device_id_type=pl.DeviceIdType.MESH,
                ).wait_recv()
            for b in range(B):
                for h in range(HQ):
                    bh = b * HQ + h
                    parts = [locstash[bh]] + [drecv[d, bh] for d in range(3)]
                    ms = [p[:, DH:DH + 1] for p in parts]
                    M = jnp.maximum(jnp.maximum(ms[0], ms[1]),
                                    jnp.maximum(ms[2], ms[3]))
                    S = jnp.float32(0)
                    C = jnp.float32(0)
                    for p, mi in zip(parts, ms):
                        w = jnp.exp(mi - M)
                        S = S + p[:, DH + 1:DH + 2] * w
                        C = C + p[:, 0:DH] * w
                    ctxscr[b, 0:G, h * DH:(h + 1) * DH] = C / S

        for b in range(B):
            out_ref[b] = jnp.dot(ctxscr[b], wo_ref[...],
                                 preferred_element_type=jnp.float32)

        @pl.when(my != N_DEV - 1)
        def _():
            for sem_i, src in enumerate([k_ref, v_ref]):
                pltpu.make_async_remote_copy(
                    src_ref=src.at[:, S_SH - HALO:S_SH, :],
                    dst_ref=kpack.at[:, C_L:C_O, :],
                    send_sem=bsend.at[sem_i], recv_sem=brecv.at[sem_i],
                    device_id=(right,), device_id_type=pl.DeviceIdType.MESH,
                ).wait_send()

        @pl.when(my != 0)
        def _():
            for sem_i, src in enumerate([k_ref, v_ref]):
                pltpu.make_async_remote_copy(
                    src_ref=src.at[:, 0:HALO, :],
                    dst_ref=kpack.at[:, C_R:W, :],
                    send_sem=bsend.at[2 + sem_i], recv_sem=brecv.at[2 + sem_i],
                    device_id=(left,), device_id_type=pl.DeviceIdType.MESH,
                ).wait_send()
            pltpu.make_async_remote_copy(
                src_ref=dbuf, dst_ref=drecv.at[0],
                send_sem=dsend.at[0], recv_sem=dsem.at[0],
                device_id=(0,), device_id_type=pl.DeviceIdType.MESH,
            ).wait_send()

        @pl.when(my == 0)
        def _():
            for d in (1, 2, 3):
                for sem_i, src in enumerate([k_ref, v_ref]):
                    pltpu.make_async_remote_copy(
                        src_ref=src.at[:, 0:G, :],
                        dst_ref=kpack.at[:, C_G:C_L, :],
                        send_sem=gsend.at[2 * (d - 1) + sem_i],
                        recv_sem=grecv.at[sem_i],
                        device_id=(d,), device_id_type=pl.DeviceIdType.MESH,
                    ).wait_send()
                pltpu.make_async_remote_copy(
                    src_ref=qscr.at[:, 0:G, :], dst_ref=q0buf,
                    send_sem=qsend.at[d - 1], recv_sem=qrecv.at[0],
                    device_id=(d,), device_id_type=pl.DeviceIdType.MESH,
                ).wait_send()

        import functools

        @functools.partial(pl.run_scoped,
                           second_barrier=pltpu.SemaphoreType.REGULAR)
        def _(second_barrier):
            for ofs in (1, 2, 3):
                pl.semaphore_signal(
                    second_barrier, inc=1,
                    device_id=(lax.rem(my + ofs, N_DEV),),
                    device_id_type=pl.DeviceIdType.MESH,
                )
            pl.semaphore_wait(second_barrier, 3)

    return pl.pallas_call(
        body,
        out_shape=jax.ShapeDtypeStruct((B, S_SH, DM), jnp.float32),
        in_specs=[pl.BlockSpec(memory_space=pltpu.VMEM)] * 5,
        out_specs=pl.BlockSpec(memory_space=pltpu.VMEM),
        scratch_shapes=[
            pltpu.VMEM((BH, W, DH), jnp.bfloat16),
            pltpu.VMEM((BH, W, DH), jnp.bfloat16),
            pltpu.VMEM((B, S_SH, HD), jnp.float32),
            pltpu.VMEM((B, G, HD), jnp.float32),
            pltpu.VMEM((BH, G, 128), jnp.float32),
            pltpu.VMEM((3, BH, G, 128), jnp.float32),
            pltpu.VMEM((BH, G, 128), jnp.float32),
            pltpu.VMEM((B, S_SH, HD), jnp.float32),
            pltpu.SemaphoreType.DMA((2,)),
            pltpu.SemaphoreType.DMA((4,)),
            pltpu.SemaphoreType.DMA((4,)),
            pltpu.SemaphoreType.DMA((6,)),
            pltpu.SemaphoreType.DMA((2,)),
            pltpu.SemaphoreType.DMA((3,)),
            pltpu.SemaphoreType.DMA((1,)),
            pltpu.SemaphoreType.DMA((1,)),
            pltpu.SemaphoreType.DMA((3,)),
        ],
        compiler_params=pltpu.CompilerParams(collective_id=0),
    )(x, Wq, Kt, Vt, Wo)


# device time: 23504 ns/iter; 1.3444x vs baseline; 1.0338x over previous
import jax
import jax.numpy as jnp
from jax import lax
from jax.experimental import pallas as pl
from jax.experimental.pallas import tpu as pltpu

N_DEV = 4
B = 2
S_SH = 256
HQ = 4
DH = 64
BH = B * HQ
DM = 512
HD = HQ * DH

G = 32
HALO = 128
RW = G + 2 * HALO
SENT = 100000

R_G, R_L, R_R = 0, G, G + HALO


def kernel(x, Wq, K_ext, V_ext, Wo):
    Kt = jnp.transpose(K_ext, (0, 2, 1, 3)).reshape(BH, S_SH, DH)
    Kt = Kt.astype(jnp.bfloat16)
    Vt = jnp.transpose(V_ext, (0, 2, 1, 3)).reshape(BH, S_SH, DH)
    Vt = Vt.astype(jnp.bfloat16)

    def body(x_ref, wq_ref, k_ref, v_ref, wo_ref, out_ref,
             krem, vrem, qscr, q0buf, dbuf, drecv, locstash, ctxscr,
             bsend, brecv, gsend, grecv, qsend, qrecv, dsend, dsem):
        my = lax.axis_index("i")
        right = lax.rem(my + 1, N_DEV)
        left = lax.rem(my + 3, N_DEV)

        vrem[...] = jnp.zeros((BH, RW, DH), jnp.bfloat16)

        barrier_sem = pltpu.get_barrier_semaphore()

        @pl.when((my == 0) | (my == 2))
        def _():
            for ofs in (1, 2, 3):
                pl.semaphore_signal(
                    barrier_sem, inc=1,
                    device_id=(lax.rem(my + ofs, N_DEV),),
                    device_id_type=pl.DeviceIdType.MESH)
            pl.semaphore_wait(barrier_sem, 3)

        @pl.when((my == 1) | (my == 3))
        def _():
            for ofs in (1, 3):
                pl.semaphore_signal(
                    barrier_sem, inc=1,
                    device_id=(lax.rem(my + ofs, N_DEV),),
                    device_id_type=pl.DeviceIdType.MESH)
            pl.semaphore_wait(barrier_sem, 2)

        @pl.when(my != N_DEV - 1)
        def _():
            for sem_i, (src, dstbuf) in enumerate(
                    [(k_ref, krem), (v_ref, vrem)]):
                pltpu.make_async_remote_copy(
                    src_ref=src.at[:, S_SH - HALO:S_SH, :],
                    dst_ref=dstbuf.at[:, R_L:R_R, :],
                    send_sem=bsend.at[sem_i],
                    recv_sem=brecv.at[sem_i],
                    device_id=(right,),
                    device_id_type=pl.DeviceIdType.MESH,
                ).start()

        @pl.when(my != 0)
        def _():
            for sem_i, (src, dstbuf) in enumerate(
                    [(k_ref, krem), (v_ref, vrem)]):
                pltpu.make_async_remote_copy(
                    src_ref=src.at[:, 0:HALO, :],
                    dst_ref=dstbuf.at[:, R_R:RW, :],
                    send_sem=bsend.at[2 + sem_i],
                    recv_sem=brecv.at[2 + sem_i],
                    device_id=(left,),
                    device_id_type=pl.DeviceIdType.MESH,
                ).start()

        @pl.when(my == 0)
        def _():
            for d in (1, 2, 3):
                for sem_i, (src, dstbuf) in enumerate(
                        [(k_ref, krem), (v_ref, vrem)]):
                    pltpu.make_async_remote_copy(
                        src_ref=src.at[:, 0:G, :],
                        dst_ref=dstbuf.at[:, R_G:R_L, :],
                        send_sem=gsend.at[2 * (d - 1) + sem_i],
                        recv_sem=grecv.at[sem_i],
                        device_id=(d,),
                        device_id_type=pl.DeviceIdType.MESH,
                    ).start()

        for b in range(B):
            qscr[b] = jnp.dot(x_ref[b], wq_ref[...],
                              preferred_element_type=jnp.float32)

        @pl.when(my == 0)
        def _():
            for d in (1, 2, 3):
                pltpu.make_async_remote_copy(
                    src_ref=qscr.at[:, 0:G, :],
                    dst_ref=q0buf,
                    send_sem=qsend.at[d - 1],
                    recv_sem=qrecv.at[0],
                    device_id=(d,),
                    device_id_type=pl.DeviceIdType.MESH,
                ).start()

        @pl.when(my != 0)
        def _():
            pltpu.make_async_remote_copy(
                src_ref=qscr.at[:, 0:G, :], dst_ref=q0buf,
                send_sem=qsend.at[0], recv_sem=qrecv.at[0],
                device_id=(0,), device_id_type=pl.DeviceIdType.MESH,
            ).wait_recv()
            for b in range(B):
                for h in range(HQ):
                    bh = b * HQ + h
                    qd = q0buf[b, :, h * DH:(h + 1) * DH]
                    sc = lax.dot_general(
                        qd.astype(jnp.bfloat16), k_ref[bh],
                        (((1,), (1,)), ((), ())),
                        preferred_element_type=jnp.float32) * 0.125
                    md = jnp.max(sc, axis=1, keepdims=True)
                    e = jnp.exp(sc - md)
                    sd = jnp.sum(e, axis=1, keepdims=True)
                    cd = jnp.dot(e.astype(jnp.bfloat16), v_ref[bh],
                                 preferred_element_type=jnp.float32)
                    dbuf[bh] = jnp.concatenate(
                        [cd, md, sd, jnp.zeros((G, 62), jnp.float32)], axis=1)
            pltpu.make_async_remote_copy(
                src_ref=dbuf, dst_ref=drecv.at[my - 1],
                send_sem=dsend.at[0], recv_sem=dsem.at[my - 1],
                device_id=(0,), device_id_type=pl.DeviceIdType.MESH,
            ).start()

        neg = jnp.float32(-1e9)
        qgA = lax.broadcasted_iota(jnp.int32, (S_SH, S_SH), 0) + my * S_SH
        gkA = lax.broadcasted_iota(jnp.int32, (S_SH, S_SH), 1) + my * S_SH
        maskA = ((jnp.abs(qgA - gkA) <= HALO) | (gkA < G)
                 | ((qgA < G) & (gkA < S_SH)))
        qgB = lax.broadcasted_iota(jnp.int32, (S_SH, RW), 0) + my * S_SH
        ci = lax.broadcasted_iota(jnp.int32, (S_SH, RW), 1)
        gkB = jnp.where(
            ci < R_L, ci + jnp.where(my == 0, SENT, 0),
            jnp.where(
                ci < R_R, ci - R_L + jnp.where(my == 0, SENT,
                                               my * S_SH - HALO),
                ci - R_R + jnp.where(my == N_DEV - 1, SENT,
                                     my * S_SH + S_SH)))
        maskB = ((jnp.abs(qgB - gkB) <= HALO) | (gkB < G)
                 | ((qgB < G) & (gkB < S_SH)))

        qbs, mAs, sAs, cAs = [], [], [], []
        for b in range(B):
            for h in range(HQ):
                bh = b * HQ + h
                qb = qscr[b, :, h * DH:(h + 1) * DH].astype(jnp.bfloat16)
                scA = lax.dot_general(
                    qb, k_ref[bh], (((1,), (1,)), ((), ())),
                    preferred_element_type=jnp.float32) * 0.125
                scA = jnp.where(maskA, scA, neg)
                mA = jnp.max(scA, axis=1, keepdims=True)
                eA = jnp.exp(scA - mA)
                sA = jnp.sum(eA, axis=1, keepdims=True)
                cA = jnp.dot(eA.astype(jnp.bfloat16), v_ref[bh],
                             preferred_element_type=jnp.float32)
                qbs.append(qb)
                mAs.append(mA)
                sAs.append(sA)
                cAs.append(cA)

        @pl.when(my != 0)
        def _():
            for sem_i, (src, dstbuf) in enumerate(
                    [(k_ref, krem), (v_ref, vrem)]):
                pltpu.make_async_remote_copy(
                    src_ref=src.at[:, S_SH - HALO:S_SH, :],
                    dst_ref=dstbuf.at[:, R_L:R_R, :],
                    send_sem=bsend.at[sem_i], recv_sem=brecv.at[sem_i],
                    device_id=(left,), device_id_type=pl.DeviceIdType.MESH,
                ).wait_recv()
                pltpu.make_async_remote_copy(
                    src_ref=src.at[:, 0:G, :],
                    dst_ref=dstbuf.at[:, R_G:R_L, :],
                    send_sem=gsend.at[sem_i], recv_sem=grecv.at[sem_i],
                    device_id=(0,), device_id_type=pl.DeviceIdType.MESH,
                ).wait_recv()

        @pl.when(my != N_DEV - 1)
        def _():
            for sem_i, (src, dstbuf) in enumerate(
                    [(k_ref, krem), (v_ref, vrem)]):
                pltpu.make_async_remote_copy(
                    src_ref=src.at[:, 0:HALO, :],
                    dst_ref=dstbuf.at[:, R_R:RW, :],
                    send_sem=bsend.at[2 + sem_i], recv_sem=brecv.at[2 + sem_i],
                    device_id=(right,), device_id_type=pl.DeviceIdType.MESH,
                ).wait_recv()

        for b in range(B):
            for h in range(HQ):
                bh = b * HQ + h
                scB = lax.dot_general(
                    qbs[bh], krem[bh], (((1,), (1,)), ((), ())),
                    preferred_element_type=jnp.float32) * 0.125
                scB = jnp.where(maskB, scB, neg)
                mB = jnp.max(scB, axis=1, keepdims=True)
                eB = jnp.exp(scB - mB)
                sB = jnp.sum(eB, axis=1, keepdims=True)
                cB = jnp.dot(eB.astype(jnp.bfloat16), vrem[bh],
                             preferred_element_type=jnp.float32)
                M = jnp.maximum(mAs[bh], mB)
                wA = jnp.exp(mAs[bh] - M)
                wB = jnp.exp(mB - M)
                s = sAs[bh] * wA + sB * wB
                c = cAs[bh] * wA + cB * wB
                ctxscr[b, :, h * DH:(h + 1) * DH] = c / s
                locstash[bh] = jnp.concatenate(
                    [c[0:G], M[0:G], s[0:G],
                     jnp.zeros((G, 62), jnp.float32)], axis=1)

        @pl.when(my == 0)
        def _():
            for d in range(3):
                pltpu.make_async_remote_copy(
                    src_ref=dbuf, dst_ref=drecv.at[d],
                    send_sem=dsend.at[0], recv_sem=dsem.at[d],
                    device_id=(0,), device_id_type=pl.DeviceIdType.MESH,
                ).wait_recv()
            for b in range(B):
                for h in range(HQ):
                    bh = b * HQ + h
                    parts = [locstash[bh]] + [drecv[d, bh] for d in range(3)]
                    ms = [p[:, DH:DH + 1] for p in parts]
                    M = jnp.maximum(jnp.maximum(ms[0], ms[1]),
                                    jnp.maximum(ms[2], ms[3]))
                    S = jnp.float32(0)
                    C = jnp.float32(0)
                    for p, mi in zip(parts, ms):
                        w = jnp.exp(mi - M)
                        S = S + p[:, DH + 1:DH + 2] * w
                        C = C + p[:, 0:DH] * w
                    ctxscr[b, 0:G, h * DH:(h + 1) * DH] = C / S

        for b in range(B):
            out_ref[b] = jnp.dot(ctxscr[b], wo_ref[...],
                                 preferred_element_type=jnp.float32)

        @pl.when(my != N_DEV - 1)
        def _():
            for sem_i, src in enumerate([k_ref, v_ref]):
                pltpu.make_async_remote_copy(
                    src_ref=src.at[:, S_SH - HALO:S_SH, :],
                    dst_ref=krem.at[:, R_L:R_R, :],
                    send_sem=bsend.at[sem_i], recv_sem=brecv.at[sem_i],
                    device_id=(right,), device_id_type=pl.DeviceIdType.MESH,
                ).wait_send()

        @pl.when(my != 0)
        def _():
            for sem_i, src in enumerate([k_ref, v_ref]):
                pltpu.make_async_remote_copy(
                    src_ref=src.at[:, 0:HALO, :],
                    dst_ref=krem.at[:, R_R:RW, :],
                    send_sem=bsend.at[2 + sem_i], recv_sem=brecv.at[2 + sem_i],
                    device_id=(left,), device_id_type=pl.DeviceIdType.MESH,
                ).wait_send()
            pltpu.make_async_remote_copy(
                src_ref=dbuf, dst_ref=drecv.at[0],
                send_sem=dsend.at[0], recv_sem=dsem.at[0],
                device_id=(0,), device_id_type=pl.DeviceIdType.MESH,
            ).wait_send()

        @pl.when(my == 0)
        def _():
            for d in (1, 2, 3):
                for sem_i, src in enumerate([k_ref, v_ref]):
                    pltpu.make_async_remote_copy(
                        src_ref=src.at[:, 0:G, :],
                        dst_ref=krem.at[:, R_G:R_L, :],
                        send_sem=gsend.at[2 * (d - 1) + sem_i],
                        recv_sem=grecv.at[sem_i],
                        device_id=(d,), device_id_type=pl.DeviceIdType.MESH,
                    ).wait_send()
                pltpu.make_async_remote_copy(
                    src_ref=qscr.at[:, 0:G, :], dst_ref=q0buf,
                    send_sem=qsend.at[d - 1], recv_sem=qrecv.at[0],
                    device_id=(d,), device_id_type=pl.DeviceIdType.MESH,
                ).wait_send()

    return pl.pallas_call(
        body,
        out_shape=jax.ShapeDtypeStruct((B, S_SH, DM), jnp.float32),
        in_specs=[pl.BlockSpec(memory_space=pltpu.VMEM)] * 5,
        out_specs=pl.BlockSpec(memory_space=pltpu.VMEM),
        scratch_shapes=[
            pltpu.VMEM((BH, RW, DH), jnp.bfloat16),
            pltpu.VMEM((BH, RW, DH), jnp.bfloat16),
            pltpu.VMEM((B, S_SH, HD), jnp.float32),
            pltpu.VMEM((B, G, HD), jnp.float32),
            pltpu.VMEM((BH, G, 128), jnp.float32),
            pltpu.VMEM((3, BH, G, 128), jnp.float32),
            pltpu.VMEM((BH, G, 128), jnp.float32),
            pltpu.VMEM((B, S_SH, HD), jnp.float32),
            pltpu.SemaphoreType.DMA((4,)),
            pltpu.SemaphoreType.DMA((4,)),
            pltpu.SemaphoreType.DMA((6,)),
            pltpu.SemaphoreType.DMA((2,)),
            pltpu.SemaphoreType.DMA((3,)),
            pltpu.SemaphoreType.DMA((1,)),
            pltpu.SemaphoreType.DMA((1,)),
            pltpu.SemaphoreType.DMA((3,)),
        ],
        compiler_params=pltpu.CompilerParams(collective_id=0),
    )(x, Wq, Kt, Vt, Wo)
